# baseline (speedup 1.0000x reference)
import jax
import jax.numpy as jnp
from jax import lax
from jax.experimental import pallas as pl
from jax.experimental.pallas import tpu as pltpu
from jax.experimental.pallas import tpu_sc as plsc

_T = 8192
_D = 768
_NC = 2
_RPC = _T // _NC
_NCH = 8
_CH = _RPC // _NCH


def _make_sc_copy():
    mesh = plsc.ScalarSubcoreMesh(axis_name="c", num_cores=_NC)

    def body(wpe_hbm, out_hbm, sems):
        cid = lax.axis_index("c")
        base = cid * _RPC

        def cp(i):
            return pltpu.make_async_copy(
                wpe_hbm.at[pl.ds(base + i * _CH, _CH)],
                out_hbm.at[pl.ds(base + i * _CH, _CH)],
                sems.at[i])

        for i in range(_NCH):
            cp(i).start()
        for i in range(_NCH):
            cp(i).wait()

    return pl.kernel(
        body,
        out_type=jax.ShapeDtypeStruct((_T, _D), jnp.float32),
        mesh=mesh,
        scratch_types=[
            pltpu.SemaphoreType.DMA((_NCH,)),
        ],
    )


_sc_copy = _make_sc_copy()


def kernel(idx, wpe):
    del idx
    return _sc_copy(wpe)


# speedup vs baseline: 21.4530x; 21.4530x over previous
import jax
import jax.numpy as jnp
from jax import lax
from jax.experimental import pallas as pl
from jax.experimental.pallas import tpu as pltpu
from jax.experimental.pallas import tpu_sc as plsc

_T = 8192
_D = 768
_NC = 2
_RPC = _T // _NC
_CH = 256
_NCH = _RPC // _CH
_NBUF = 8
_LAG = 4


def _make_sc_copy():
    mesh = plsc.ScalarSubcoreMesh(axis_name="c", num_cores=_NC)

    def body(wpe_hbm, out_hbm, buf, rsems, wsems):
        cid = lax.axis_index("c")
        base = cid * _RPC

        def rd(i, slot):
            return pltpu.make_async_copy(
                wpe_hbm.at[pl.ds(base + i * _CH, _CH)], buf.at[slot],
                rsems.at[slot])

        def wr(i, slot):
            return pltpu.make_async_copy(
                buf.at[slot], out_hbm.at[pl.ds(base + i * _CH, _CH)],
                wsems.at[slot])

        for j in range(_NBUF - _LAG):
            rd(j, j).start()

        def step(i, carry):
            @pl.when(i >= _LAG)
            def _():
                wr(i - _LAG, lax.rem(i - _LAG, _NBUF)).wait()

            nxt = i + _NBUF - _LAG

            @pl.when(nxt < _NCH)
            def _():
                rd(nxt, lax.rem(nxt, _NBUF)).start()

            slot = lax.rem(i, _NBUF)
            rd(i, slot).wait()
            wr(i, slot).start()
            return carry

        lax.fori_loop(0, _NCH, step, 0, unroll=False)

        def drain(i, carry):
            wr(i, lax.rem(i, _NBUF)).wait()
            return carry

        lax.fori_loop(max(0, _NCH - _LAG), _NCH, drain, 0, unroll=False)

    return pl.kernel(
        body,
        out_type=jax.ShapeDtypeStruct((_T, _D), jnp.float32),
        mesh=mesh,
        scratch_types=[
            pltpu.MemorySpace.VMEM_SHARED((_NBUF, _CH, _D), jnp.float32),
            pltpu.SemaphoreType.DMA((_NBUF,)),
            pltpu.SemaphoreType.DMA((_NBUF,)),
        ],
    )


_sc_copy = _make_sc_copy()


def kernel(idx, wpe):
    del idx
    return _sc_copy(wpe)
